# baseline (speedup 1.0000x reference)
import functools

import jax
import jax.numpy as jnp
from jax import lax
from jax.experimental import pallas as pl
from jax.experimental.pallas import tpu as pltpu
from jax.experimental.pallas import tpu_sc as plsc

VOCAB = 100000
EMB = 64
B = 4096
L = 200

NW = 32
NP = B // 2
PPW = NP // NW
RB = 2 * L * EMB // 128


def _body(xp_hbm, emb_hbm, pos_hbm, out_hbm,
          idx0, idx1, rows0, rows1, pos_sh,
          sem_p0, sem_p1, sem_i0, sem_i1, sem_g, sem_s0, sem_s1):
    idx_v = (idx0, idx1)
    rows_v = (rows0, rows1)
    sem_p = (sem_p0, sem_p1)
    sem_i = (sem_i0, sem_i1)
    sem_s = (sem_s0, sem_s1)

    sid = lax.axis_index("s")
    wid = sid * 2 + lax.axis_index("c")
    base = wid * PPW
    end = base + PPW

    @pl.when(sid == 0)
    def _init():
        pltpu.sync_copy(pos_hbm, pos_sh)

    plsc.subcore_barrier()

    def start_load(p, b):
        pltpu.async_copy(pos_sh, rows_v[b], sem_p[b])
        pltpu.async_copy(xp_hbm.at[p], idx_v[b], sem_i[b])

    def wait_load(b):
        pltpu.make_async_copy(pos_sh, rows_v[b], sem_p[b]).wait()
        pltpu.make_async_copy(xp_hbm.at[0], idx_v[b], sem_i[b]).wait()

    def wait_store(b):
        pltpu.make_async_copy(rows_v[b], out_hbm.at[0], sem_s[b]).wait()

    start_load(base, 0)
    pltpu.async_copy(pos_sh, rows_v[1], sem_s1)

    @pl.loop(0, PPW, step=2)
    def _pair(t):
        for db in range(2):
            p = base + t + db
            b = db
            o = 1 - db

            wait_store(o)
            start_load(lax.min(p + 1, end - 1), o)

            wait_load(b)
            g0 = pltpu.async_copy(
                emb_hbm.at[idx_v[b].at[0]], rows_v[b], sem_g, add=True)
            g1 = pltpu.async_copy(
                emb_hbm.at[idx_v[b].at[1]], rows_v[b], sem_g, add=True)
            g0.wait()
            g1.wait()
            pltpu.async_copy(rows_v[b], out_hbm.at[p], sem_s[b])

    wait_store(1)
    wait_load(0)


@jax.jit
def kernel(x, emb_table, pos_table):
    z = jnp.zeros((VOCAB, EMB), jnp.float32)
    embcat = jnp.concatenate(
        [
            jnp.concatenate([emb_table, z], axis=1),
            jnp.concatenate([z, emb_table], axis=1),
        ],
        axis=0,
    )
    ev = x[:, 0::2].reshape(NP, L)
    od = x[:, 1::2].reshape(NP, L) + VOCAB
    xp = jnp.stack([ev, od], axis=1)
    posb = jnp.tile(jnp.reshape(pos_table[:L], (L * EMB // 128, 128)), (2, 1))

    mesh = plsc.VectorSubcoreMesh(core_axis_name="c", subcore_axis_name="s")
    out = pl.kernel(
        _body,
        out_type=jax.ShapeDtypeStruct((NP, RB, 128), jnp.float32),
        mesh=mesh,
        compiler_params=pltpu.CompilerParams(use_tc_tiling_on_sc=False),
        scratch_types=[
            pltpu.VMEM((2, L), jnp.int32),
            pltpu.VMEM((2, L), jnp.int32),
            pltpu.VMEM((RB, 128), jnp.float32),
            pltpu.VMEM((RB, 128), jnp.float32),
            pltpu.VMEM_SHARED((RB, 128), jnp.float32),
            pltpu.SemaphoreType.DMA,
            pltpu.SemaphoreType.DMA,
            pltpu.SemaphoreType.DMA,
            pltpu.SemaphoreType.DMA,
            pltpu.SemaphoreType.DMA,
            pltpu.SemaphoreType.DMA,
            pltpu.SemaphoreType.DMA,
        ],
    )(xp, embcat, posb)
    return jnp.reshape(out, (B, L, EMB))
